# Initial kernel scaffold; baseline (speedup 1.0000x reference)
import jax
import jax.numpy as jnp
from jax.experimental import pallas as pl


def kernel(x, edge_index, batch, W1, b1, g1, be1, W2, b2, g2, be2, W3, b3, g3, be3, W4, b4, g4, be4, fcW, fcb):
    raise NotImplementedError("write your pallas kernel here")


# speedup vs baseline: 19.0794x; 19.0794x over previous
import functools

import jax
import jax.numpy as jnp
from jax import lax
from jax.experimental import pallas as pl
from jax.experimental.pallas import tpu as pltpu
from jax.experimental.pallas import tpu_sc as plsc

N = 10000
E = 320000
F_IN = 128
H = 64
C = 10
G = 128
EPS = 1e-5

NC = 2
NS = 16
NW = NC * NS

K = 128
NK = (E + NW * K - 1) // (NW * K)
E_PAD = NW * NK * K
N_ACC = 10112
RPT = N_ACC // NS
DW = 16

@functools.cache
def _sc_mesh():
    return plsc.VectorSubcoreMesh(
        core_axis_name="c", subcore_axis_name="s",
        num_cores=NC, num_subcores=NS)


def _deg_body(dst_hbm, out_hbm, idx_v, ones_v, buf_v, acc_sh, sem):
    del sem
    cid = lax.axis_index("c")
    sid = lax.axis_index("s")
    wid = cid * NS + sid

    def fill_ones(i, _):
        ones_v[i, :] = jnp.ones((DW,), jnp.float32)
        return 0

    lax.fori_loop(0, K, fill_ones, 0)

    def fill_zero(i, _):
        buf_v[i, :] = jnp.zeros((DW,), jnp.float32)
        return 0

    lax.fori_loop(0, RPT, fill_zero, 0)
    pltpu.sync_copy(buf_v, acc_sh.at[pl.ds(sid * RPT, RPT)])
    pltpu.sync_copy(dst_hbm.at[wid], idx_v)
    plsc.subcore_barrier()

    def body(j, _):
        pltpu.sync_copy(ones_v, acc_sh.at[idx_v.at[j]], add=True)
        return 0

    lax.fori_loop(0, NK, body, 0)
    plsc.subcore_barrier()
    pltpu.sync_copy(acc_sh.at[pl.ds(sid * RPT, RPT)], buf_v)
    pltpu.sync_copy(buf_v, out_hbm.at[cid, pl.ds(sid * RPT, RPT)])


@functools.cache
def _deg_call():
    return pl.kernel(
        _deg_body,
        out_type=jax.ShapeDtypeStruct((NC, N_ACC, DW), jnp.float32),
        mesh=_sc_mesh(),
        compiler_params=pltpu.CompilerParams(use_tc_tiling_on_sc=False),
        scratch_types=[
            pltpu.VMEM((NK, K), jnp.int32),
            pltpu.VMEM((K, DW), jnp.float32),
            pltpu.VMEM((RPT, DW), jnp.float32),
            pltpu.VMEM_SHARED((N_ACC, DW), jnp.float32),
            pltpu.SemaphoreType.DMA,
        ],
    )


def _layer_body(hp_hbm, src_hbm, dst_hbm, out_hbm,
                srcv, dstv, bufa, bufb, initv, acc_sh, sema, semb):
    cid = lax.axis_index("c")
    sid = lax.axis_index("s")
    wid = cid * NS + sid

    pltpu.sync_copy(hp_hbm.at[pl.ds(sid * RPT, RPT)], initv)
    pltpu.sync_copy(initv, acc_sh.at[pl.ds(sid * RPT, RPT)])
    pltpu.sync_copy(src_hbm.at[wid], srcv)
    pltpu.sync_copy(dst_hbm.at[wid], dstv)
    plsc.subcore_barrier()

    pltpu.async_copy(hp_hbm.at[srcv.at[0]], bufa, sema)

    def body(i, _):
        j = 2 * i
        pltpu.async_copy(hp_hbm.at[srcv.at[j + 1]], bufb, semb)
        pltpu.make_async_copy(hp_hbm.at[srcv.at[0]], bufa, sema).wait()
        pltpu.sync_copy(bufa, acc_sh.at[dstv.at[j]], add=True)
        pltpu.async_copy(hp_hbm.at[srcv.at[j + 2]], bufa, sema)
        pltpu.make_async_copy(hp_hbm.at[srcv.at[0]], bufb, semb).wait()
        pltpu.sync_copy(bufb, acc_sh.at[dstv.at[j + 1]], add=True)
        return 0

    lax.fori_loop(0, (NK - 1) // 2, body, 0)
    pltpu.make_async_copy(hp_hbm.at[srcv.at[0]], bufa, sema).wait()
    pltpu.sync_copy(bufa, acc_sh.at[dstv.at[NK - 1]], add=True)
    plsc.subcore_barrier()
    pltpu.sync_copy(acc_sh.at[pl.ds(sid * RPT, RPT)], initv)
    pltpu.sync_copy(initv, out_hbm.at[cid, pl.ds(sid * RPT, RPT)])


@functools.cache
def _layer_call():
    return pl.kernel(
        _layer_body,
        out_type=jax.ShapeDtypeStruct((NC, N_ACC, H), jnp.float32),
        mesh=_sc_mesh(),
        compiler_params=pltpu.CompilerParams(use_tc_tiling_on_sc=False),
        scratch_types=[
            pltpu.VMEM((NK, K), jnp.int32),
            pltpu.VMEM((NK, K), jnp.int32),
            pltpu.VMEM((K, H), jnp.float32),
            pltpu.VMEM((K, H), jnp.float32),
            pltpu.VMEM((RPT, H), jnp.float32),
            pltpu.VMEM_SHARED((N_ACC, H), jnp.float32),
            pltpu.SemaphoreType.DMA,
            pltpu.SemaphoreType.DMA,
        ],
    )


def _prep_body(pdeg_ref, xp_ref, w1_ref, dinv_ref, hp_ref):
    deg = pdeg_ref[0, :, 0:1] + pdeg_ref[1, :, 0:1] + 1.0
    dinv = lax.rsqrt(deg)
    dinv_ref[...] = dinv
    z = jnp.dot(xp_ref[...], w1_ref[...], preferred_element_type=jnp.float32)
    hp_ref[...] = z * dinv


def _bn_relu(p_ref, hp_ref, dinv_ref, b_ref, g_ref, be_ref):
    dinv = dinv_ref[...]
    s = p_ref[0] + p_ref[1] - hp_ref[...]
    z = s * dinv + b_ref[...]
    mask = lax.broadcasted_iota(jnp.int32, (N_ACC, 1), 0) < N
    mean = jnp.sum(jnp.where(mask, z, 0.0), axis=0, keepdims=True) / N
    d = z - mean
    var = jnp.sum(jnp.where(mask, d * d, 0.0), axis=0, keepdims=True) / N
    y = g_ref[...] * d * lax.rsqrt(var + EPS) + be_ref[...]
    return jnp.maximum(y, 0.0), dinv


def _mid_body(p_ref, hp_ref, dinv_ref, b_ref, g_ref, be_ref, w_ref, out_ref):
    y, dinv = _bn_relu(p_ref, hp_ref, dinv_ref, b_ref, g_ref, be_ref)
    out_ref[...] = jnp.dot(
        y, w_ref[...], preferred_element_type=jnp.float32) * dinv


def _fin_body(p_ref, hp_ref, dinv_ref, b_ref, g_ref, be_ref, batch_ref,
              fcw_ref, fcb_ref, out_ref):
    y, _ = _bn_relu(p_ref, hp_ref, dinv_ref, b_ref, g_ref, be_ref)
    bt = batch_ref[...]
    gids = lax.broadcasted_iota(jnp.int32, (G, N_ACC), 0)
    oh = jnp.where(gids == bt, 1.0, 0.0)
    sums = jnp.dot(oh, y, preferred_element_type=jnp.float32)
    counts = jnp.sum(oh, axis=1, keepdims=True)
    pooled = sums / jnp.maximum(counts, 1.0)
    logits = jnp.dot(
        pooled, fcw_ref[...], preferred_element_type=jnp.float32) + fcb_ref[...]
    m = jnp.max(logits, axis=1, keepdims=True)
    lse = jnp.log(jnp.sum(jnp.exp(logits - m), axis=1, keepdims=True)) + m
    out_ref[...] = logits - lse


_prep_call = pl.pallas_call(
    _prep_body,
    out_shape=(
        jax.ShapeDtypeStruct((N_ACC, 1), jnp.float32),
        jax.ShapeDtypeStruct((N_ACC, H), jnp.float32),
    ),
)

_mid_call = pl.pallas_call(
    _mid_body,
    out_shape=jax.ShapeDtypeStruct((N_ACC, H), jnp.float32),
)

_fin_call = pl.pallas_call(
    _fin_body,
    out_shape=jax.ShapeDtypeStruct((G, C), jnp.float32),
)


def kernel(x, edge_index, batch, W1, b1, g1, be1, W2, b2, g2, be2,
           W3, b3, g3, be3, W4, b4, g4, be4, fcW, fcb):
    pad = E_PAD - E
    src = jnp.concatenate(
        [edge_index[0], jnp.zeros((pad,), jnp.int32)]).reshape(NW, NK, K)
    dst = jnp.concatenate(
        [edge_index[1], jnp.full((pad,), N, jnp.int32)]).reshape(NW, NK, K)
    xp = jnp.pad(x, ((0, N_ACC - N), (0, 0)))
    batchp = jnp.pad(batch, (0, N_ACC - N),
                     constant_values=G).reshape(1, N_ACC)

    pdeg = _deg_call()(dst)
    dinv, hp = _prep_call(pdeg, xp, W1)

    for b, g, be, Wn in ((b1, g1, be1, W2), (b2, g2, be2, W3),
                         (b3, g3, be3, W4)):
        p = _layer_call()(hp, src, dst)
        hp = _mid_call(p, hp, dinv, b.reshape(1, H), g.reshape(1, H),
                       be.reshape(1, H), Wn)

    p = _layer_call()(hp, src, dst)
    return _fin_call(p, hp, dinv, b4.reshape(1, H), g4.reshape(1, H),
                     be4.reshape(1, H), batchp, fcW, fcb)
